# baseline (speedup 1.0000x reference)
import jax
import jax.numpy as jnp
from jax import lax
from jax.experimental import pallas as pl
from jax.experimental.pallas import tpu as pltpu

_N = 32768
_D = 32
_B = 16
_TOPK = 1024
_FC0 = 256
_W = 2112
_ALIGN = 64
_RCH = 256
_SCH = 4096
_PCH = 256
_KCH = 2048
_EPS_BN = 1e-5
_EPS_NORM = 1e-12


def _topk_body(starts_ref, length_ref, x_ref, w_ref, scal_ref, flat_ref, stats_ref):
    i = pl.program_id(0)
    conv_b = scal_ref[0]
    bn1_g = scal_ref[1]
    bn1_b = scal_ref[2]

    @pl.when(i == 0)
    def _():
        w = w_ref[...]
        y = lax.dot_general(w, x_ref[...], (((0,), (1,)), ((), ())),
                            preferred_element_type=jnp.float32)
        q = lax.dot_general(y, y, (((1,), (1,)), ((), ())),
                            preferred_element_type=jnp.float32)
        ma = jnp.sum(y, axis=1, keepdims=True) * (1.0 / _N)
        var = jnp.maximum(q * (1.0 / _N) - ma * ma, 0.0)
        stats_ref[...] = jnp.concatenate([ma + conv_b, var], axis=1)

    @pl.when(i > 0)
    def _seg():
        _topk_segment(starts_ref, length_ref, x_ref, w_ref, scal_ref,
                      flat_ref, stats_ref, i - 1)


def _topk_segment(starts_ref, length_ref, x_ref, w_ref, scal_ref,
                  flat_ref, stats_ref, seg):
    conv_b = scal_ref[0]
    bn1_g = scal_ref[1]
    bn1_b = scal_ref[2]

    s = starts_ref[seg]
    seg_len = length_ref[seg]
    w0 = (s // _ALIGN) * _ALIGN
    r = s - w0

    x_win = x_ref[pl.ds(w0, _W), :]
    w = w_ref[...]
    att_row = lax.dot_general(w, x_win, (((0,), (1,)), ((), ())),
                              preferred_element_type=jnp.float32)
    att_row = att_row + conv_b
    m = stats_ref[0:1, 0:1]
    v = stats_ref[0:1, 1:2]
    a_row = bn1_g * (att_row - m) / jnp.sqrt(v + _EPS_BN) + bn1_b

    col = lax.broadcasted_iota(jnp.int32, (1, _W), 1)
    valid = (col >= r) & (col < r + seg_len)
    a_row = jnp.where(valid, a_row, -jnp.inf)
    a_col = a_row.reshape(_W, 1)

    colsum = jnp.zeros((1, _W), jnp.float32)
    corr_parts = []
    for c0 in range(0, _W, _RCH):
        rows = min(_RCH, _W - c0)
        end = c0 + rows
        a_blk = a_col[c0:end]
        a_diag = a_row[:, c0:end]
        km = lax.broadcasted_iota(jnp.int32, (rows, rows), 0)
        jm = lax.broadcasted_iota(jnp.int32, (rows, rows), 1)
        diag = (a_blk > a_diag) | ((a_blk == a_diag) & (km < jm))
        diag_f = jnp.where(diag, 1.0, 0.0)
        dsum_col = jnp.sum(diag_f, axis=0, keepdims=True)
        parts = [dsum_col]
        if end < _W:
            a_off = a_row[:, end:]
            off_f = jnp.where(a_blk >= a_off, 1.0, 0.0)
            parts.append(jnp.sum(off_f, axis=0, keepdims=True))
            osum_row = jnp.sum(off_f, axis=1, keepdims=True)
            corr_parts.append(jnp.float32(_W - end) - osum_row)
        else:
            corr_parts.append(jnp.zeros((rows, 1), jnp.float32))
        if c0 > 0:
            parts.insert(0, jnp.zeros((1, c0), jnp.float32))
        colsum = colsum + jnp.concatenate(parts, axis=1)
    corr_col = jnp.concatenate(corr_parts, axis=0)
    rank = colsum + corr_col.reshape(1, _W)

    for p0 in range(0, _TOPK, _PCH):
        rv = (lax.broadcasted_iota(jnp.int32, (_PCH, 1), 0) + p0).astype(jnp.float32)
        p_blk = jnp.where(rank == rv, 1.0, 0.0)
        fb = lax.dot_general(p_blk, x_win, (((1,), (0,)), ((), ())),
                             preferred_element_type=jnp.float32)
        flat_ref[0, pl.ds(p0, _PCH), :] = fb


def _fc_body(flat_ref, fcw_ref, fcb_ref, g_ref, b_ref, out_ref, acc_ref):
    k = pl.program_id(0)
    nk = pl.num_programs(0)

    @pl.when(k == 0)
    def _():
        acc_ref[...] = jnp.zeros((_B, _FC0), jnp.float32)

    acc_ref[...] += lax.dot_general(flat_ref[...], fcw_ref[...],
                                    (((1,), (1,)), ((), ())),
                                    preferred_element_type=jnp.float32)

    @pl.when(k == nk - 1)
    def _():
        h = acc_ref[...] + fcb_ref[...]
        m2 = jnp.mean(h, axis=0, keepdims=True)
        v2 = jnp.mean((h - m2) ** 2, axis=0, keepdims=True)
        hn = g_ref[...] * (h - m2) / jnp.sqrt(v2 + _EPS_BN) + b_ref[...]
        nrm = jnp.sqrt(jnp.sum(hn * hn, axis=1, keepdims=True))
        out_ref[...] = hn / jnp.maximum(nrm, _EPS_NORM)


def kernel(x, length, conv_w, conv_b, bn1_g, bn1_b, fc_w, fc_b, bn2_g, bn2_b):
    length = length.astype(jnp.int32)
    ends = jnp.cumsum(length)
    starts = (ends - length).astype(jnp.int32)
    scal = jnp.concatenate([conv_b, bn1_g, bn1_b]).astype(jnp.float32)
    wcol = conv_w.reshape(_D, 1).astype(jnp.float32)

    flat = pl.pallas_call(
        _topk_body,
        grid=(_B + 1,),
        in_specs=[
            pl.BlockSpec(memory_space=pltpu.SMEM),
            pl.BlockSpec(memory_space=pltpu.SMEM),
            pl.BlockSpec((_N, _D), lambda i: (0, 0)),
            pl.BlockSpec((_D, 1), lambda i: (0, 0)),
            pl.BlockSpec(memory_space=pltpu.SMEM),
        ],
        out_specs=pl.BlockSpec((1, _TOPK, _D),
                               lambda i: (jnp.maximum(i - 1, 0), 0, 0)),
        out_shape=jax.ShapeDtypeStruct((_B, _TOPK, _D), jnp.float32),
        scratch_shapes=[pltpu.VMEM((1, 2), jnp.float32)],
    )(starts, length, x, wcol, scal)

    flat2 = flat.reshape(_B, _TOPK * _D)
    fcb2 = fc_b.reshape(1, _FC0).astype(jnp.float32)
    g2 = bn2_g.reshape(1, _FC0).astype(jnp.float32)
    b2 = bn2_b.reshape(1, _FC0).astype(jnp.float32)
    nk = (_TOPK * _D) // _KCH

    out = pl.pallas_call(
        _fc_body,
        grid=(nk,),
        in_specs=[
            pl.BlockSpec((_B, _KCH), lambda k: (0, k)),
            pl.BlockSpec((_FC0, _KCH), lambda k: (0, k)),
            pl.BlockSpec((1, _FC0), lambda k: (0, 0)),
            pl.BlockSpec((1, _FC0), lambda k: (0, 0)),
            pl.BlockSpec((1, _FC0), lambda k: (0, 0)),
        ],
        out_specs=pl.BlockSpec((_B, _FC0), lambda k: (0, 0)),
        out_shape=jax.ShapeDtypeStruct((_B, _FC0), jnp.float32),
        scratch_shapes=[pltpu.VMEM((_B, _FC0), jnp.float32)],
    )(flat2, fc_w, fcb2, g2, b2)
    return out


# speedup vs baseline: 1.0312x; 1.0312x over previous
import jax
import jax.numpy as jnp
from jax import lax
from jax.experimental import pallas as pl
from jax.experimental.pallas import tpu as pltpu

_N = 32768
_D = 32
_B = 16
_TOPK = 1024
_FC0 = 256
_W = 2112
_ALIGN = 64
_RCH = 512
_SCH = 4096
_PCH = 512
_KCH = 2048
_EPS_BN = 1e-5
_EPS_NORM = 1e-12


def _topk_body(starts_ref, length_ref, x_ref, w_ref, scal_ref, flat_ref, stats_ref):
    i = pl.program_id(0)
    conv_b = scal_ref[0]
    bn1_g = scal_ref[1]
    bn1_b = scal_ref[2]

    @pl.when(i == 0)
    def _():
        w = w_ref[...]
        y = lax.dot_general(w, x_ref[...], (((0,), (1,)), ((), ())),
                            preferred_element_type=jnp.float32)
        q = lax.dot_general(y, y, (((1,), (1,)), ((), ())),
                            preferred_element_type=jnp.float32)
        ma = jnp.sum(y, axis=1, keepdims=True) * (1.0 / _N)
        var = jnp.maximum(q * (1.0 / _N) - ma * ma, 0.0)
        stats_ref[...] = jnp.concatenate([ma + conv_b, var], axis=1)

    @pl.when(i > 0)
    def _seg():
        _topk_segment(starts_ref, length_ref, x_ref, w_ref, scal_ref,
                      flat_ref, stats_ref, i - 1)


def _topk_segment(starts_ref, length_ref, x_ref, w_ref, scal_ref,
                  flat_ref, stats_ref, seg):
    conv_b = scal_ref[0]
    bn1_g = scal_ref[1]
    bn1_b = scal_ref[2]

    s = starts_ref[seg]
    seg_len = length_ref[seg]
    w0 = (s // _ALIGN) * _ALIGN
    r = s - w0

    x_win = x_ref[pl.ds(w0, _W), :]
    w = w_ref[...]
    att_row = lax.dot_general(w, x_win, (((0,), (1,)), ((), ())),
                              preferred_element_type=jnp.float32)
    att_row = att_row + conv_b
    m = stats_ref[0:1, 0:1]
    v = stats_ref[0:1, 1:2]
    a_row = bn1_g * (att_row - m) / jnp.sqrt(v + _EPS_BN) + bn1_b

    col = lax.broadcasted_iota(jnp.int32, (1, _W), 1)
    valid = (col >= r) & (col < r + seg_len)
    a_row = jnp.where(valid, a_row, -jnp.inf)
    a_col = a_row.reshape(_W, 1)

    colsum = jnp.zeros((1, _W), jnp.float32)
    corr_parts = []
    for c0 in range(0, _W, _RCH):
        rows = min(_RCH, _W - c0)
        end = c0 + rows
        a_blk = a_col[c0:end]
        a_diag = a_row[:, c0:end]
        km = lax.broadcasted_iota(jnp.int32, (rows, rows), 0)
        jm = lax.broadcasted_iota(jnp.int32, (rows, rows), 1)
        diag = (a_blk > a_diag) | ((a_blk == a_diag) & (km < jm))
        diag_f = jnp.where(diag, 1.0, 0.0)
        dsum_col = jnp.sum(diag_f, axis=0, keepdims=True)
        parts = [dsum_col]
        if end < _W:
            a_off = a_row[:, end:]
            off_f = jnp.where(a_blk >= a_off, 1.0, 0.0)
            parts.append(jnp.sum(off_f, axis=0, keepdims=True))
            osum_row = jnp.sum(off_f, axis=1, keepdims=True)
            corr_parts.append(jnp.float32(_W - end) - osum_row)
        else:
            corr_parts.append(jnp.zeros((rows, 1), jnp.float32))
        if c0 > 0:
            parts.insert(0, jnp.zeros((1, c0), jnp.float32))
        colsum = colsum + jnp.concatenate(parts, axis=1)
    corr_col = jnp.concatenate(corr_parts, axis=0)
    rank = colsum + corr_col.reshape(1, _W)

    for p0 in range(0, _TOPK, _PCH):
        rv = (lax.broadcasted_iota(jnp.int32, (_PCH, 1), 0) + p0).astype(jnp.float32)
        p_blk = jnp.where(rank == rv, 1.0, 0.0)
        fb = lax.dot_general(p_blk, x_win, (((1,), (0,)), ((), ())),
                             preferred_element_type=jnp.float32)
        flat_ref[0, pl.ds(p0, _PCH), :] = fb


def _fc_body(flat_ref, fcw_ref, fcb_ref, g_ref, b_ref, out_ref, acc_ref):
    k = pl.program_id(0)
    nk = pl.num_programs(0)

    @pl.when(k == 0)
    def _():
        acc_ref[...] = jnp.zeros((_B, _FC0), jnp.float32)

    acc_ref[...] += lax.dot_general(flat_ref[...], fcw_ref[...],
                                    (((1,), (1,)), ((), ())),
                                    preferred_element_type=jnp.float32)

    @pl.when(k == nk - 1)
    def _():
        h = acc_ref[...] + fcb_ref[...]
        m2 = jnp.mean(h, axis=0, keepdims=True)
        v2 = jnp.mean((h - m2) ** 2, axis=0, keepdims=True)
        hn = g_ref[...] * (h - m2) / jnp.sqrt(v2 + _EPS_BN) + b_ref[...]
        nrm = jnp.sqrt(jnp.sum(hn * hn, axis=1, keepdims=True))
        out_ref[...] = hn / jnp.maximum(nrm, _EPS_NORM)


def kernel(x, length, conv_w, conv_b, bn1_g, bn1_b, fc_w, fc_b, bn2_g, bn2_b):
    length = length.astype(jnp.int32)
    ends = jnp.cumsum(length)
    starts = (ends - length).astype(jnp.int32)
    scal = jnp.concatenate([conv_b, bn1_g, bn1_b]).astype(jnp.float32)
    wcol = conv_w.reshape(_D, 1).astype(jnp.float32)

    flat = pl.pallas_call(
        _topk_body,
        grid=(_B + 1,),
        in_specs=[
            pl.BlockSpec(memory_space=pltpu.SMEM),
            pl.BlockSpec(memory_space=pltpu.SMEM),
            pl.BlockSpec((_N, _D), lambda i: (0, 0)),
            pl.BlockSpec((_D, 1), lambda i: (0, 0)),
            pl.BlockSpec(memory_space=pltpu.SMEM),
        ],
        out_specs=pl.BlockSpec((1, _TOPK, _D),
                               lambda i: (jnp.maximum(i - 1, 0), 0, 0)),
        out_shape=jax.ShapeDtypeStruct((_B, _TOPK, _D), jnp.float32),
        scratch_shapes=[pltpu.VMEM((1, 2), jnp.float32)],
    )(starts, length, x, wcol, scal)

    flat2 = flat.reshape(_B, _TOPK * _D)
    fcb2 = fc_b.reshape(1, _FC0).astype(jnp.float32)
    g2 = bn2_g.reshape(1, _FC0).astype(jnp.float32)
    b2 = bn2_b.reshape(1, _FC0).astype(jnp.float32)
    nk = (_TOPK * _D) // _KCH

    out = pl.pallas_call(
        _fc_body,
        grid=(nk,),
        in_specs=[
            pl.BlockSpec((_B, _KCH), lambda k: (0, k)),
            pl.BlockSpec((_FC0, _KCH), lambda k: (0, k)),
            pl.BlockSpec((1, _FC0), lambda k: (0, 0)),
            pl.BlockSpec((1, _FC0), lambda k: (0, 0)),
            pl.BlockSpec((1, _FC0), lambda k: (0, 0)),
        ],
        out_specs=pl.BlockSpec((_B, _FC0), lambda k: (0, 0)),
        out_shape=jax.ShapeDtypeStruct((_B, _FC0), jnp.float32),
        scratch_shapes=[pltpu.VMEM((_B, _FC0), jnp.float32)],
    )(flat2, fc_w, fcb2, g2, b2)
    return out


# speedup vs baseline: 1.0761x; 1.0435x over previous
import jax
import jax.numpy as jnp
from jax import lax
from jax.experimental import pallas as pl
from jax.experimental.pallas import tpu as pltpu

_N = 32768
_D = 32
_B = 16
_TOPK = 1024
_FC0 = 256
_W = 2112
_ALIGN = 64
_RCH = 512
_SCH = 4096
_PCH = 1024
_KCH = 4096
_EPS_BN = 1e-5
_EPS_NORM = 1e-12


def _topk_body(starts_ref, length_ref, x_ref, w_ref, scal_ref, flat_ref, stats_ref):
    i = pl.program_id(0)
    conv_b = scal_ref[0]
    bn1_g = scal_ref[1]
    bn1_b = scal_ref[2]

    @pl.when(i == 0)
    def _():
        w = w_ref[...]
        y = lax.dot_general(w, x_ref[...], (((0,), (1,)), ((), ())),
                            preferred_element_type=jnp.float32)
        q = lax.dot_general(y, y, (((1,), (1,)), ((), ())),
                            preferred_element_type=jnp.float32)
        ma = jnp.sum(y, axis=1, keepdims=True) * (1.0 / _N)
        var = jnp.maximum(q * (1.0 / _N) - ma * ma, 0.0)
        stats_ref[...] = jnp.concatenate([ma + conv_b, var], axis=1)

    @pl.when(i > 0)
    def _seg():
        _topk_segment(starts_ref, length_ref, x_ref, w_ref, scal_ref,
                      flat_ref, stats_ref, i - 1)


def _topk_segment(starts_ref, length_ref, x_ref, w_ref, scal_ref,
                  flat_ref, stats_ref, seg):
    conv_b = scal_ref[0]
    bn1_g = scal_ref[1]
    bn1_b = scal_ref[2]

    s = starts_ref[seg]
    seg_len = length_ref[seg]
    w0 = (s // _ALIGN) * _ALIGN
    r = s - w0

    x_win = x_ref[pl.ds(w0, _W), :]
    w = w_ref[...]
    att_row = lax.dot_general(w, x_win, (((0,), (1,)), ((), ())),
                              preferred_element_type=jnp.float32)
    att_row = att_row + conv_b
    m = stats_ref[0:1, 0:1]
    v = stats_ref[0:1, 1:2]
    a_row = bn1_g * (att_row - m) / jnp.sqrt(v + _EPS_BN) + bn1_b

    col = lax.broadcasted_iota(jnp.int32, (1, _W), 1)
    valid = (col >= r) & (col < r + seg_len)
    a_row = jnp.where(valid, a_row, -jnp.inf)
    a_col = a_row.reshape(_W, 1)

    colsum = jnp.zeros((1, _W), jnp.float32)
    corr_parts = []
    for c0 in range(0, _W, _RCH):
        rows = min(_RCH, _W - c0)
        end = c0 + rows
        a_blk = a_col[c0:end]
        a_diag = a_row[:, c0:end]
        km = lax.broadcasted_iota(jnp.int32, (rows, rows), 0)
        jm = lax.broadcasted_iota(jnp.int32, (rows, rows), 1)
        diag = (a_blk > a_diag) | ((a_blk == a_diag) & (km < jm))
        diag_f = jnp.where(diag, 1.0, 0.0)
        dsum_col = jnp.sum(diag_f, axis=0, keepdims=True)
        parts = [dsum_col]
        if end < _W:
            a_off = a_row[:, end:]
            off_f = jnp.where(a_blk >= a_off, 1.0, 0.0)
            parts.append(jnp.sum(off_f, axis=0, keepdims=True))
            osum_row = jnp.sum(off_f, axis=1, keepdims=True)
            corr_parts.append(jnp.float32(_W - end) - osum_row)
        else:
            corr_parts.append(jnp.zeros((rows, 1), jnp.float32))
        if c0 > 0:
            parts.insert(0, jnp.zeros((1, c0), jnp.float32))
        colsum = colsum + jnp.concatenate(parts, axis=1)
    corr_col = jnp.concatenate(corr_parts, axis=0)
    rank = colsum + corr_col.reshape(1, _W)

    for p0 in range(0, _TOPK, _PCH):
        rv = (lax.broadcasted_iota(jnp.int32, (_PCH, 1), 0) + p0).astype(jnp.float32)
        p_blk = jnp.where(rank == rv, 1.0, 0.0)
        fb = lax.dot_general(p_blk, x_win, (((1,), (0,)), ((), ())),
                             preferred_element_type=jnp.float32)
        flat_ref[0, pl.ds(p0, _PCH), :] = fb


def _fc_body(flat_ref, fcw_ref, fcb_ref, g_ref, b_ref, out_ref, acc_ref):
    k = pl.program_id(0)
    nk = pl.num_programs(0)

    @pl.when(k == 0)
    def _():
        acc_ref[...] = jnp.zeros((_B, _FC0), jnp.float32)

    acc_ref[...] += lax.dot_general(flat_ref[...], fcw_ref[...],
                                    (((1,), (1,)), ((), ())),
                                    preferred_element_type=jnp.float32)

    @pl.when(k == nk - 1)
    def _():
        h = acc_ref[...] + fcb_ref[...]
        m2 = jnp.mean(h, axis=0, keepdims=True)
        v2 = jnp.mean((h - m2) ** 2, axis=0, keepdims=True)
        hn = g_ref[...] * (h - m2) / jnp.sqrt(v2 + _EPS_BN) + b_ref[...]
        nrm = jnp.sqrt(jnp.sum(hn * hn, axis=1, keepdims=True))
        out_ref[...] = hn / jnp.maximum(nrm, _EPS_NORM)


def kernel(x, length, conv_w, conv_b, bn1_g, bn1_b, fc_w, fc_b, bn2_g, bn2_b):
    length = length.astype(jnp.int32)
    ends = jnp.cumsum(length)
    starts = (ends - length).astype(jnp.int32)
    scal = jnp.concatenate([conv_b, bn1_g, bn1_b]).astype(jnp.float32)
    wcol = conv_w.reshape(_D, 1).astype(jnp.float32)

    flat = pl.pallas_call(
        _topk_body,
        grid=(_B + 1,),
        in_specs=[
            pl.BlockSpec(memory_space=pltpu.SMEM),
            pl.BlockSpec(memory_space=pltpu.SMEM),
            pl.BlockSpec((_N, _D), lambda i: (0, 0)),
            pl.BlockSpec((_D, 1), lambda i: (0, 0)),
            pl.BlockSpec(memory_space=pltpu.SMEM),
        ],
        out_specs=pl.BlockSpec((1, _TOPK, _D),
                               lambda i: (jnp.maximum(i - 1, 0), 0, 0)),
        out_shape=jax.ShapeDtypeStruct((_B, _TOPK, _D), jnp.float32),
        scratch_shapes=[pltpu.VMEM((1, 2), jnp.float32)],
    )(starts, length, x, wcol, scal)

    flat2 = flat.reshape(_B, _TOPK * _D)
    fcb2 = fc_b.reshape(1, _FC0).astype(jnp.float32)
    g2 = bn2_g.reshape(1, _FC0).astype(jnp.float32)
    b2 = bn2_b.reshape(1, _FC0).astype(jnp.float32)
    nk = (_TOPK * _D) // _KCH

    out = pl.pallas_call(
        _fc_body,
        grid=(nk,),
        in_specs=[
            pl.BlockSpec((_B, _KCH), lambda k: (0, k)),
            pl.BlockSpec((_FC0, _KCH), lambda k: (0, k)),
            pl.BlockSpec((1, _FC0), lambda k: (0, 0)),
            pl.BlockSpec((1, _FC0), lambda k: (0, 0)),
            pl.BlockSpec((1, _FC0), lambda k: (0, 0)),
        ],
        out_specs=pl.BlockSpec((_B, _FC0), lambda k: (0, 0)),
        out_shape=jax.ShapeDtypeStruct((_B, _FC0), jnp.float32),
        scratch_shapes=[pltpu.VMEM((_B, _FC0), jnp.float32)],
    )(flat2, fc_w, fcb2, g2, b2)
    return out


# speedup vs baseline: 1.0777x; 1.0015x over previous
import jax
import jax.numpy as jnp
from jax import lax
from jax.experimental import pallas as pl
from jax.experimental.pallas import tpu as pltpu

_N = 32768
_D = 32
_B = 16
_TOPK = 1024
_FC0 = 256
_W = 2112
_ALIGN = 64
_RCH = 512
_SCH = 4096
_PCH = 1024
_KCH = 4096
_EPS_BN = 1e-5
_EPS_NORM = 1e-12


def _topk_body(starts_ref, length_ref, x_ref, w_ref, scal_ref, flat_ref, stats_ref):
    i = pl.program_id(0)
    conv_b = scal_ref[0]
    bn1_g = scal_ref[1]
    bn1_b = scal_ref[2]

    @pl.when(i == 0)
    def _():
        w = w_ref[...]
        y = lax.dot_general(w, x_ref[...], (((0,), (1,)), ((), ())),
                            preferred_element_type=jnp.float32)
        q = lax.dot_general(y, y, (((1,), (1,)), ((), ())),
                            preferred_element_type=jnp.float32)
        ma = jnp.sum(y, axis=1, keepdims=True) * (1.0 / _N)
        var = jnp.maximum(q * (1.0 / _N) - ma * ma, 0.0)
        stats_ref[...] = jnp.concatenate([ma + conv_b, var], axis=1)

    @pl.when(i > 0)
    def _seg():
        _topk_segment(starts_ref, length_ref, x_ref, w_ref, scal_ref,
                      flat_ref, stats_ref, i - 1)


def _topk_segment(starts_ref, length_ref, x_ref, w_ref, scal_ref,
                  flat_ref, stats_ref, seg):
    conv_b = scal_ref[0]
    bn1_g = scal_ref[1]
    bn1_b = scal_ref[2]

    s = starts_ref[seg]
    seg_len = length_ref[seg]
    w0 = (s // _ALIGN) * _ALIGN
    r = s - w0

    x_win = x_ref[pl.ds(w0, _W), :]
    w = w_ref[...]
    att_row = lax.dot_general(w, x_win, (((0,), (1,)), ((), ())),
                              preferred_element_type=jnp.float32)
    att_row = att_row + conv_b
    m = stats_ref[0:1, 0:1]
    v = stats_ref[0:1, 1:2]
    a_row = bn1_g * (att_row - m) / jnp.sqrt(v + _EPS_BN) + bn1_b

    col = lax.broadcasted_iota(jnp.int32, (1, _W), 1)
    valid = (col >= r) & (col < r + seg_len)
    a_row = jnp.where(valid, a_row, -jnp.inf)
    a_col = a_row.reshape(_W, 1)

    colsum = jnp.zeros((1, _W), jnp.float32)
    corr_parts = []
    for c0 in range(0, _W, _RCH):
        rows = min(_RCH, _W - c0)
        end = c0 + rows
        a_blk = a_col[c0:end]
        a_diag = a_row[:, c0:end]
        km = lax.broadcasted_iota(jnp.int32, (rows, rows), 0)
        jm = lax.broadcasted_iota(jnp.int32, (rows, rows), 1)
        diag = (a_blk > a_diag) | ((a_blk == a_diag) & (km < jm))
        diag_f = jnp.where(diag, 1.0, 0.0)
        dsum_col = jnp.sum(diag_f, axis=0, keepdims=True)
        parts = [dsum_col]
        if end < _W:
            a_off = a_row[:, end:]
            off_f = jnp.where(a_blk >= a_off, 1.0, 0.0)
            parts.append(jnp.sum(off_f, axis=0, keepdims=True))
            osum_row = jnp.sum(off_f, axis=1, keepdims=True)
            corr_parts.append(jnp.float32(_W - end) - osum_row)
        else:
            corr_parts.append(jnp.zeros((rows, 1), jnp.float32))
        if c0 > 0:
            parts.insert(0, jnp.zeros((1, c0), jnp.float32))
        colsum = colsum + jnp.concatenate(parts, axis=1)
    corr_col = jnp.concatenate(corr_parts, axis=0)
    rank = colsum + corr_col.reshape(1, _W)

    x_win_bf = x_win.astype(jnp.bfloat16)
    for p0 in range(0, _TOPK, _PCH):
        rv = (lax.broadcasted_iota(jnp.int32, (_PCH, 1), 0) + p0).astype(jnp.float32)
        p_blk = jnp.where(rank == rv, 1.0, 0.0).astype(jnp.bfloat16)
        fb = lax.dot_general(p_blk, x_win_bf, (((1,), (0,)), ((), ())),
                             preferred_element_type=jnp.float32)
        flat_ref[0, pl.ds(p0, _PCH), :] = fb.astype(jnp.bfloat16)


def _fc_body(flat_ref, fcw_ref, fcb_ref, g_ref, b_ref, out_ref, acc_ref):
    k = pl.program_id(0)
    nk = pl.num_programs(0)

    @pl.when(k == 0)
    def _():
        acc_ref[...] = jnp.zeros((_B, _FC0), jnp.float32)

    acc_ref[...] += lax.dot_general(flat_ref[...],
                                    fcw_ref[...].astype(jnp.bfloat16),
                                    (((1,), (1,)), ((), ())),
                                    preferred_element_type=jnp.float32)

    @pl.when(k == nk - 1)
    def _():
        h = acc_ref[...] + fcb_ref[...]
        m2 = jnp.mean(h, axis=0, keepdims=True)
        v2 = jnp.mean((h - m2) ** 2, axis=0, keepdims=True)
        hn = g_ref[...] * (h - m2) / jnp.sqrt(v2 + _EPS_BN) + b_ref[...]
        nrm = jnp.sqrt(jnp.sum(hn * hn, axis=1, keepdims=True))
        out_ref[...] = hn / jnp.maximum(nrm, _EPS_NORM)


def kernel(x, length, conv_w, conv_b, bn1_g, bn1_b, fc_w, fc_b, bn2_g, bn2_b):
    length = length.astype(jnp.int32)
    ends = jnp.cumsum(length)
    starts = (ends - length).astype(jnp.int32)
    scal = jnp.concatenate([conv_b, bn1_g, bn1_b]).astype(jnp.float32)
    wcol = conv_w.reshape(_D, 1).astype(jnp.float32)

    flat = pl.pallas_call(
        _topk_body,
        grid=(_B + 1,),
        in_specs=[
            pl.BlockSpec(memory_space=pltpu.SMEM),
            pl.BlockSpec(memory_space=pltpu.SMEM),
            pl.BlockSpec((_N, _D), lambda i: (0, 0)),
            pl.BlockSpec((_D, 1), lambda i: (0, 0)),
            pl.BlockSpec(memory_space=pltpu.SMEM),
        ],
        out_specs=pl.BlockSpec((1, _TOPK, _D),
                               lambda i: (jnp.maximum(i - 1, 0), 0, 0)),
        out_shape=jax.ShapeDtypeStruct((_B, _TOPK, _D), jnp.bfloat16),
        scratch_shapes=[pltpu.VMEM((1, 2), jnp.float32)],
    )(starts, length, x, wcol, scal)

    flat2 = flat.reshape(_B, _TOPK * _D)
    fcb2 = fc_b.reshape(1, _FC0).astype(jnp.float32)
    g2 = bn2_g.reshape(1, _FC0).astype(jnp.float32)
    b2 = bn2_b.reshape(1, _FC0).astype(jnp.float32)
    nk = (_TOPK * _D) // _KCH

    out = pl.pallas_call(
        _fc_body,
        grid=(nk,),
        in_specs=[
            pl.BlockSpec((_B, _KCH), lambda k: (0, k)),
            pl.BlockSpec((_FC0, _KCH), lambda k: (0, k)),
            pl.BlockSpec((1, _FC0), lambda k: (0, 0)),
            pl.BlockSpec((1, _FC0), lambda k: (0, 0)),
            pl.BlockSpec((1, _FC0), lambda k: (0, 0)),
        ],
        out_specs=pl.BlockSpec((_B, _FC0), lambda k: (0, 0)),
        out_shape=jax.ShapeDtypeStruct((_B, _FC0), jnp.float32),
        scratch_shapes=[pltpu.VMEM((_B, _FC0), jnp.float32)],
    )(flat2, fc_w, fcb2, g2, b2)
    return out


# speedup vs baseline: 1.0895x; 1.0109x over previous
import jax
import jax.numpy as jnp
from jax import lax
from jax.experimental import pallas as pl
from jax.experimental.pallas import tpu as pltpu

_N = 32768
_D = 32
_B = 16
_TOPK = 1024
_FC0 = 256
_W = 2112
_ALIGN = 64
_RCH = 512
_SCH = 4096
_PCH = 1024
_KCH = 8192
_EPS_BN = 1e-5
_EPS_NORM = 1e-12


def _topk_body(starts_ref, length_ref, x_ref, w_ref, scal_ref, flat_ref, stats_ref):
    i = pl.program_id(0)
    conv_b = scal_ref[0]
    bn1_g = scal_ref[1]
    bn1_b = scal_ref[2]

    @pl.when(i == 0)
    def _():
        w = w_ref[...]
        y = lax.dot_general(w, x_ref[...], (((0,), (1,)), ((), ())),
                            preferred_element_type=jnp.float32)
        q = lax.dot_general(y, y, (((1,), (1,)), ((), ())),
                            preferred_element_type=jnp.float32)
        ma = jnp.sum(y, axis=1, keepdims=True) * (1.0 / _N)
        var = jnp.maximum(q * (1.0 / _N) - ma * ma, 0.0)
        stats_ref[...] = jnp.concatenate([ma + conv_b, var], axis=1)

    @pl.when(i > 0)
    def _seg():
        _topk_segment(starts_ref, length_ref, x_ref, w_ref, scal_ref,
                      flat_ref, stats_ref, i - 1)


def _topk_segment(starts_ref, length_ref, x_ref, w_ref, scal_ref,
                  flat_ref, stats_ref, seg):
    conv_b = scal_ref[0]
    bn1_g = scal_ref[1]
    bn1_b = scal_ref[2]

    s = starts_ref[seg]
    seg_len = length_ref[seg]
    w0 = (s // _ALIGN) * _ALIGN
    r = s - w0

    x_win = x_ref[pl.ds(w0, _W), :]
    w = w_ref[...]
    att_row = lax.dot_general(w, x_win, (((0,), (1,)), ((), ())),
                              preferred_element_type=jnp.float32)
    att_row = att_row + conv_b
    m = stats_ref[0:1, 0:1]
    v = stats_ref[0:1, 1:2]
    a_row = bn1_g * (att_row - m) / jnp.sqrt(v + _EPS_BN) + bn1_b

    col = lax.broadcasted_iota(jnp.int32, (1, _W), 1)
    valid = (col >= r) & (col < r + seg_len)
    a_row = jnp.where(valid, a_row, -jnp.inf)
    a_col = a_row.reshape(_W, 1)

    colsum = jnp.zeros((1, _W), jnp.float32)
    corr_parts = []
    for c0 in range(0, _W, _RCH):
        rows = min(_RCH, _W - c0)
        end = c0 + rows
        a_blk = a_col[c0:end]
        a_diag = a_row[:, c0:end]
        km = lax.broadcasted_iota(jnp.int32, (rows, rows), 0)
        jm = lax.broadcasted_iota(jnp.int32, (rows, rows), 1)
        diag = (a_blk > a_diag) | ((a_blk == a_diag) & (km < jm))
        diag_f = jnp.where(diag, 1.0, 0.0)
        dsum_col = jnp.sum(diag_f, axis=0, keepdims=True)
        parts = [dsum_col]
        if end < _W:
            a_off = a_row[:, end:]
            off_f = jnp.where(a_blk >= a_off, 1.0, 0.0)
            parts.append(jnp.sum(off_f, axis=0, keepdims=True))
            osum_row = jnp.sum(off_f, axis=1, keepdims=True)
            corr_parts.append(jnp.float32(_W - end) - osum_row)
        else:
            corr_parts.append(jnp.zeros((rows, 1), jnp.float32))
        if c0 > 0:
            parts.insert(0, jnp.zeros((1, c0), jnp.float32))
        colsum = colsum + jnp.concatenate(parts, axis=1)
    corr_col = jnp.concatenate(corr_parts, axis=0)
    rank = colsum + corr_col.reshape(1, _W)

    for p0 in range(0, _TOPK, _PCH):
        rv = (lax.broadcasted_iota(jnp.int32, (_PCH, 1), 0) + p0).astype(jnp.float32)
        p_blk = jnp.where(rank == rv, 1.0, 0.0)
        fb = lax.dot_general(p_blk, x_win, (((1,), (0,)), ((), ())),
                             preferred_element_type=jnp.float32)
        flat_ref[0, pl.ds(p0, _PCH), :] = fb


def _fc_body(flat_ref, fcw_ref, fcb_ref, g_ref, b_ref, out_ref, acc_ref):
    k = pl.program_id(0)
    nk = pl.num_programs(0)

    @pl.when(k == 0)
    def _():
        acc_ref[...] = jnp.zeros((_B, _FC0), jnp.float32)

    acc_ref[...] += lax.dot_general(flat_ref[...], fcw_ref[...],
                                    (((1,), (1,)), ((), ())),
                                    preferred_element_type=jnp.float32)

    @pl.when(k == nk - 1)
    def _():
        h = acc_ref[...] + fcb_ref[...]
        m2 = jnp.mean(h, axis=0, keepdims=True)
        v2 = jnp.mean((h - m2) ** 2, axis=0, keepdims=True)
        hn = g_ref[...] * (h - m2) / jnp.sqrt(v2 + _EPS_BN) + b_ref[...]
        nrm = jnp.sqrt(jnp.sum(hn * hn, axis=1, keepdims=True))
        out_ref[...] = hn / jnp.maximum(nrm, _EPS_NORM)


def kernel(x, length, conv_w, conv_b, bn1_g, bn1_b, fc_w, fc_b, bn2_g, bn2_b):
    length = length.astype(jnp.int32)
    ends = jnp.cumsum(length)
    starts = (ends - length).astype(jnp.int32)
    scal = jnp.concatenate([conv_b, bn1_g, bn1_b]).astype(jnp.float32)
    wcol = conv_w.reshape(_D, 1).astype(jnp.float32)

    flat = pl.pallas_call(
        _topk_body,
        grid=(_B + 1,),
        in_specs=[
            pl.BlockSpec(memory_space=pltpu.SMEM),
            pl.BlockSpec(memory_space=pltpu.SMEM),
            pl.BlockSpec((_N, _D), lambda i: (0, 0)),
            pl.BlockSpec((_D, 1), lambda i: (0, 0)),
            pl.BlockSpec(memory_space=pltpu.SMEM),
        ],
        out_specs=pl.BlockSpec((1, _TOPK, _D),
                               lambda i: (jnp.maximum(i - 1, 0), 0, 0)),
        out_shape=jax.ShapeDtypeStruct((_B, _TOPK, _D), jnp.float32),
        scratch_shapes=[pltpu.VMEM((1, 2), jnp.float32)],
    )(starts, length, x, wcol, scal)

    flat2 = flat.reshape(_B, _TOPK * _D)
    fcb2 = fc_b.reshape(1, _FC0).astype(jnp.float32)
    g2 = bn2_g.reshape(1, _FC0).astype(jnp.float32)
    b2 = bn2_b.reshape(1, _FC0).astype(jnp.float32)
    nk = (_TOPK * _D) // _KCH

    out = pl.pallas_call(
        _fc_body,
        grid=(nk,),
        in_specs=[
            pl.BlockSpec((_B, _KCH), lambda k: (0, k)),
            pl.BlockSpec((_FC0, _KCH), lambda k: (0, k)),
            pl.BlockSpec((1, _FC0), lambda k: (0, 0)),
            pl.BlockSpec((1, _FC0), lambda k: (0, 0)),
            pl.BlockSpec((1, _FC0), lambda k: (0, 0)),
        ],
        out_specs=pl.BlockSpec((_B, _FC0), lambda k: (0, 0)),
        out_shape=jax.ShapeDtypeStruct((_B, _FC0), jnp.float32),
        scratch_shapes=[pltpu.VMEM((_B, _FC0), jnp.float32)],
    )(flat2, fc_w, fcb2, g2, b2)
    return out


# speedup vs baseline: 1.1526x; 1.0579x over previous
import jax
import jax.numpy as jnp
from jax import lax
from jax.experimental import pallas as pl
from jax.experimental.pallas import tpu as pltpu

_N = 32768
_D = 32
_B = 16
_TOPK = 1024
_FC0 = 256
_W = 2112
_ALIGN = 64
_RCH = 512
_SCH = 4096
_PCH = 1024
_KCH = 8192
_EPS_BN = 1e-5
_EPS_NORM = 1e-12


def _topk_body(starts_ref, length_ref, x_ref, w_ref, scal_ref, flat_ref, stats_ref):
    i = pl.program_id(0)
    conv_b = scal_ref[0]
    bn1_g = scal_ref[1]
    bn1_b = scal_ref[2]

    @pl.when(i == 0)
    def _():
        w = w_ref[...]
        y = lax.dot_general(w, x_ref[...], (((0,), (1,)), ((), ())),
                            preferred_element_type=jnp.float32)
        q = lax.dot_general(y, y, (((1,), (1,)), ((), ())),
                            preferred_element_type=jnp.float32)
        ma = jnp.sum(y, axis=1, keepdims=True) * (1.0 / _N)
        var = jnp.maximum(q * (1.0 / _N) - ma * ma, 0.0)
        stats_ref[...] = jnp.concatenate([ma + conv_b, var], axis=1)

    @pl.when(i > 0)
    def _seg():
        _topk_segment(starts_ref, length_ref, x_ref, w_ref, scal_ref,
                      flat_ref, stats_ref, i - 1)


def _topk_segment(starts_ref, length_ref, x_ref, w_ref, scal_ref,
                  flat_ref, stats_ref, seg):
    conv_b = scal_ref[0]
    bn1_g = scal_ref[1]
    bn1_b = scal_ref[2]

    s = starts_ref[seg]
    seg_len = length_ref[seg]
    w0 = (s // _ALIGN) * _ALIGN
    r = s - w0

    x_win = x_ref[pl.ds(w0, _W), :]
    w = w_ref[...]
    att_row = lax.dot_general(w, x_win, (((0,), (1,)), ((), ())),
                              preferred_element_type=jnp.float32)
    att_row = att_row + conv_b
    m = stats_ref[0:1, 0:1]
    v = stats_ref[0:1, 1:2]
    a_row = bn1_g * (att_row - m) / jnp.sqrt(v + _EPS_BN) + bn1_b

    col = lax.broadcasted_iota(jnp.int32, (1, _W), 1)
    valid = (col >= r) & (col < r + seg_len)
    a_row = jnp.where(valid, a_row, -jnp.inf)
    a_col = a_row.reshape(_W, 1)

    colsum = jnp.zeros((1, _W), jnp.float32)
    corr_parts = []
    for c0 in range(0, _W, _RCH):
        rows = min(_RCH, _W - c0)
        end = c0 + rows
        a_blk = a_col[c0:end]
        a_diag = a_row[:, c0:end]
        km = lax.broadcasted_iota(jnp.int32, (rows, rows), 0)
        jm = lax.broadcasted_iota(jnp.int32, (rows, rows), 1)
        diag = (a_blk > a_diag) | ((a_blk == a_diag) & (km < jm))
        diag_f = jnp.where(diag, 1.0, 0.0)
        dsum_col = jnp.sum(diag_f, axis=0, keepdims=True)
        parts = [dsum_col]
        if end < _W:
            a_off = a_row[:, end:]
            off_f = jnp.where(a_blk >= a_off, 1.0, 0.0)
            parts.append(jnp.sum(off_f, axis=0, keepdims=True))
            osum_row = jnp.sum(off_f, axis=1, keepdims=True)
            corr_parts.append(jnp.float32(_W - end) - osum_row)
        else:
            corr_parts.append(jnp.zeros((rows, 1), jnp.float32))
        if c0 > 0:
            parts.insert(0, jnp.zeros((1, c0), jnp.float32))
        colsum = colsum + jnp.concatenate(parts, axis=1)
    corr_col = jnp.concatenate(corr_parts, axis=0)
    rank = colsum + corr_col.reshape(1, _W)

    for p0 in range(0, _TOPK, _PCH):
        rv = (lax.broadcasted_iota(jnp.int32, (_PCH, 1), 0) + p0).astype(jnp.float32)
        p_blk = jnp.where(rank == rv, 1.0, 0.0)
        fb = lax.dot_general(p_blk, x_win, (((1,), (0,)), ((), ())),
                             preferred_element_type=jnp.float32)
        flat_ref[0, pl.ds(p0, _PCH), :] = fb


def _fc_body(flat_ref, fcw_ref, fcb_ref, g_ref, b_ref, out_ref, acc_ref):
    k = pl.program_id(0)
    nk = pl.num_programs(0)

    @pl.when(k == 0)
    def _():
        acc_ref[...] = jnp.zeros((_B, _FC0), jnp.float32)

    acc_ref[...] += lax.dot_general(flat_ref[...].reshape(_B, _KCH),
                                    fcw_ref[...],
                                    (((1,), (1,)), ((), ())),
                                    preferred_element_type=jnp.float32)

    @pl.when(k == nk - 1)
    def _():
        h = acc_ref[...] + fcb_ref[...]
        m2 = jnp.mean(h, axis=0, keepdims=True)
        v2 = jnp.mean((h - m2) ** 2, axis=0, keepdims=True)
        hn = g_ref[...] * (h - m2) / jnp.sqrt(v2 + _EPS_BN) + b_ref[...]
        nrm = jnp.sqrt(jnp.sum(hn * hn, axis=1, keepdims=True))
        out_ref[...] = hn / jnp.maximum(nrm, _EPS_NORM)


def kernel(x, length, conv_w, conv_b, bn1_g, bn1_b, fc_w, fc_b, bn2_g, bn2_b):
    length = length.astype(jnp.int32)
    ends = jnp.cumsum(length)
    starts = (ends - length).astype(jnp.int32)
    scal = jnp.concatenate([conv_b, bn1_g, bn1_b]).astype(jnp.float32)
    wcol = conv_w.reshape(_D, 1).astype(jnp.float32)

    flat = pl.pallas_call(
        _topk_body,
        grid=(_B + 1,),
        in_specs=[
            pl.BlockSpec(memory_space=pltpu.SMEM),
            pl.BlockSpec(memory_space=pltpu.SMEM),
            pl.BlockSpec((_N, _D), lambda i: (0, 0)),
            pl.BlockSpec((_D, 1), lambda i: (0, 0)),
            pl.BlockSpec(memory_space=pltpu.SMEM),
        ],
        out_specs=pl.BlockSpec((1, _TOPK, _D),
                               lambda i: (jnp.maximum(i - 1, 0), 0, 0)),
        out_shape=jax.ShapeDtypeStruct((_B, _TOPK, _D), jnp.float32),
        scratch_shapes=[pltpu.VMEM((1, 2), jnp.float32)],
    )(starts, length, x, wcol, scal)


    fcb2 = fc_b.reshape(1, _FC0).astype(jnp.float32)
    g2 = bn2_g.reshape(1, _FC0).astype(jnp.float32)
    b2 = bn2_b.reshape(1, _FC0).astype(jnp.float32)
    nk = (_TOPK * _D) // _KCH

    out = pl.pallas_call(
        _fc_body,
        grid=(nk,),
        in_specs=[
            pl.BlockSpec((_B, _KCH // _D, _D), lambda k: (0, k, 0)),
            pl.BlockSpec((_FC0, _KCH), lambda k: (0, k)),
            pl.BlockSpec((1, _FC0), lambda k: (0, 0)),
            pl.BlockSpec((1, _FC0), lambda k: (0, 0)),
            pl.BlockSpec((1, _FC0), lambda k: (0, 0)),
        ],
        out_specs=pl.BlockSpec((_B, _FC0), lambda k: (0, 0)),
        out_shape=jax.ShapeDtypeStruct((_B, _FC0), jnp.float32),
        scratch_shapes=[pltpu.VMEM((_B, _FC0), jnp.float32)],
    )(flat, fc_w, fcb2, g2, b2)
    return out
